# baseline (speedup 1.0000x reference)
import functools

import jax
import jax.numpy as jnp
from jax import lax
from jax.experimental import pallas as pl
from jax.experimental.pallas import tpu as pltpu
from jax.experimental.pallas import tpu_sc as plsc

B = 16384
D = 128
L = 16
NC = 2
NS = 16
NW = NC * NS
BPW = B // NW
CHUNK = 16
NCHUNK = BPW // CHUNK
BLOCKS_PER_CHUNK = CHUNK // L

_mesh = plsc.VectorSubcoreMesh(core_axis_name="c", subcore_axis_name="s")


@functools.partial(
    pl.kernel,
    mesh=_mesh,
    out_type=jax.ShapeDtypeStruct((B,), jnp.float32),
    compiler_params=pltpu.CompilerParams(
        needs_layout_passes=False,
        disable_bounds_checks=True,
        disable_semaphore_checks=True,
        use_tc_tiling_on_sc=False,
    ),
    scratch_types=[
        pltpu.VMEM((BPW,), jnp.int32),
        pltpu.VMEM((BPW, D), jnp.float32),
        pltpu.VMEM((D,), jnp.float32),
        pltpu.VMEM((BPW,), jnp.float32),
        pltpu.SemaphoreType.DMA,
    ],
)
def _sc_dot_gather(idx_hbm, table_hbm, user_hbm, out_hbm,
                   idx_v, rows_v, user_v, out_v, sem):
    wid = lax.axis_index("s") * NC + lax.axis_index("c")
    base = wid * BPW

    cp_idx = pltpu.async_copy(idx_hbm.at[pl.ds(base, BPW)], idx_v, sem)
    cp_user = pltpu.async_copy(user_hbm.at[0], user_v, sem)
    cp_idx.wait()
    cp_user.wait()

    for j in range(NCHUNK):
        pltpu.async_copy(
            table_hbm.at[idx_v.at[pl.ds(j * CHUNK, CHUNK)]],
            rows_v.at[pl.ds(j * CHUNK, CHUNK)],
            sem,
        )

    uvecs = [user_v[pl.ds(c * L, L)] for c in range(D // L)]
    lane = lax.iota(jnp.int32, L)

    def row_body(row, vec):
        acc = uvecs[0] * rows_v[row, pl.ds(0, L)]
        for c in range(1, D // L):
            acc = acc + uvecs[c] * rows_v[row, pl.ds(c * L, L)]
        dot = jnp.sum(acc)
        return jnp.where(lane == (row & (L - 1)), dot, vec)

    def block_body(b, carry):
        pltpu.make_async_copy(
            table_hbm.at[pl.ds(0, CHUNK)],
            rows_v.at[pl.ds(0, CHUNK)],
            sem,
        ).wait()
        vec = lax.fori_loop(b * L, (b + 1) * L, row_body,
                            jnp.zeros((L,), jnp.float32))
        out_v[pl.ds(b * L, L)] = vec
        return carry

    half = BPW // 2
    out_cps = []
    for h in range(2):
        lax.fori_loop(h * (BPW // L // 2), (h + 1) * (BPW // L // 2),
                      block_body, 0)
        out_cps.append(pltpu.async_copy(
            out_v.at[pl.ds(h * half, half)],
            out_hbm.at[pl.ds(base + h * half, half)],
            sem,
        ))
    for cp in out_cps:
        cp.wait()


def kernel(user_input, item_input, item_embedding, user_embedding):
    return _sc_dot_gather(item_input, item_embedding, user_embedding)


# speedup vs baseline: 1.0134x; 1.0134x over previous
import functools

import jax
import jax.numpy as jnp
from jax import lax
from jax.experimental import pallas as pl
from jax.experimental.pallas import tpu as pltpu
from jax.experimental.pallas import tpu_sc as plsc

B = 16384
D = 128
L = 16
NC = 2
NS = 16
NW = NC * NS
BPW = B // NW
CHUNK = 16
NCHUNK = BPW // CHUNK
BLOCKS_PER_CHUNK = CHUNK // L

_mesh = plsc.VectorSubcoreMesh(core_axis_name="c", subcore_axis_name="s")


@functools.partial(
    pl.kernel,
    mesh=_mesh,
    out_type=jax.ShapeDtypeStruct((B,), jnp.float32),
    compiler_params=pltpu.CompilerParams(
        needs_layout_passes=False,
        disable_bounds_checks=True,
        disable_semaphore_checks=True,
        use_tc_tiling_on_sc=False,
    ),
    scratch_types=[
        pltpu.VMEM((BPW,), jnp.int32),
        pltpu.VMEM((BPW, D), jnp.float32),
        pltpu.VMEM((D,), jnp.float32),
        pltpu.VMEM((BPW,), jnp.float32),
        pltpu.SemaphoreType.DMA,
    ],
)
def _sc_dot_gather(idx_hbm, table_hbm, user_hbm, out_hbm,
                   idx_v, rows_v, user_v, out_v, sem):
    wid = lax.axis_index("s") * NC + lax.axis_index("c")
    base = wid * BPW

    cp_idx = pltpu.async_copy(idx_hbm.at[pl.ds(base, BPW)], idx_v, sem)
    cp_user = pltpu.async_copy(user_hbm.at[0], user_v, sem)
    cp_idx.wait()
    cp_user.wait()

    for j in range(NCHUNK):
        pltpu.async_copy(
            table_hbm.at[idx_v.at[pl.ds(j * CHUNK, CHUNK)]],
            rows_v.at[pl.ds(j * CHUNK, CHUNK)],
            sem,
        )

    uvecs = [user_v[pl.ds(c * L, L)] for c in range(D // L)]
    lane = lax.iota(jnp.int32, L)

    def row_body(row, vec):
        acc = uvecs[0] * rows_v[row, pl.ds(0, L)]
        for c in range(1, D // L):
            acc = acc + uvecs[c] * rows_v[row, pl.ds(c * L, L)]
        dot = jnp.sum(acc)
        return jnp.where(lane == (row & (L - 1)), dot, vec)

    def block_body(b, carry):
        pltpu.make_async_copy(
            table_hbm.at[pl.ds(0, CHUNK)],
            rows_v.at[pl.ds(0, CHUNK)],
            sem,
        ).wait()
        vec = lax.fori_loop(b * L, (b + 1) * L, row_body,
                            jnp.zeros((L,), jnp.float32))
        out_v[pl.ds(b * L, L)] = vec
        return carry

    lax.fori_loop(0, BPW // L, block_body, 0)

    pltpu.sync_copy(out_v, out_hbm.at[pl.ds(base, BPW)])


def kernel(user_input, item_input, item_embedding, user_embedding):
    return _sc_dot_gather(item_input.astype(jnp.int32), item_embedding,
                          user_embedding)


# speedup vs baseline: 1.0145x; 1.0011x over previous
import functools

import jax
import jax.numpy as jnp
from jax import lax
from jax.experimental import pallas as pl
from jax.experimental.pallas import tpu as pltpu
from jax.experimental.pallas import tpu_sc as plsc

B = 16384
D = 128
L = 16
NC = 2
NS = 16
NW = NC * NS
BPW = B // NW
CHUNK = 16
NCHUNK = BPW // CHUNK

_mesh = plsc.VectorSubcoreMesh(core_axis_name="c", subcore_axis_name="s")


@functools.partial(
    pl.kernel,
    mesh=_mesh,
    out_type=jax.ShapeDtypeStruct((B,), jnp.float32),
    compiler_params=pltpu.CompilerParams(
        needs_layout_passes=False,
        disable_bounds_checks=True,
        disable_semaphore_checks=True,
        use_tc_tiling_on_sc=False,
    ),
    scratch_types=[
        pltpu.VMEM((BPW,), jnp.int32),
        pltpu.VMEM((BPW, D), jnp.float32),
        pltpu.VMEM((D,), jnp.float32),
        pltpu.VMEM((BPW,), jnp.float32),
        pltpu.SemaphoreType.DMA,
    ],
)
def _sc_dot_gather(idx_hbm, table_hbm, user_hbm, out_hbm,
                   idx_v, rows_v, user_v, out_v, sem):
    wid = lax.axis_index("s") * NC + lax.axis_index("c")
    base = wid * BPW

    cp_idx = pltpu.async_copy(idx_hbm.at[pl.ds(base, BPW)], idx_v, sem)
    cp_user = pltpu.async_copy(user_hbm.at[0], user_v, sem)
    cp_idx.wait()
    cp_user.wait()

    for j in range(NCHUNK):
        pltpu.async_copy(
            table_hbm.at[idx_v.at[pl.ds(j * CHUNK, CHUNK)]],
            rows_v.at[pl.ds(j * CHUNK, CHUNK)],
            sem,
        )

    uvecs = [user_v[pl.ds(c * L, L)] for c in range(D // L)]
    lane = lax.iota(jnp.int32, L)

    def row_body(row, vec):
        acc = uvecs[0] * rows_v[row, pl.ds(0, L)]
        for c in range(1, D // L):
            acc = acc + uvecs[c] * rows_v[row, pl.ds(c * L, L)]
        dot = jnp.sum(acc)
        return jnp.where(lane == (row & (L - 1)), dot, vec)

    def block_body(b, carry):
        pltpu.make_async_copy(
            table_hbm.at[pl.ds(0, CHUNK)],
            rows_v.at[pl.ds(0, CHUNK)],
            sem,
        ).wait()
        vec = lax.fori_loop(b * L, (b + 1) * L, row_body,
                            jnp.zeros((L,), jnp.float32))
        out_v[pl.ds(b * L, L)] = vec
        return carry

    lax.fori_loop(0, BPW // L, block_body, 0)

    pltpu.sync_copy(out_v, out_hbm.at[pl.ds(base, BPW)])


def kernel(user_input, item_input, item_embedding, user_embedding):
    return _sc_dot_gather(item_input.astype(jnp.int32), item_embedding,
                          user_embedding)
